# baseline (speedup 1.0000x reference)
import functools

import jax
import jax.numpy as jnp
from jax import lax
from jax.experimental import pallas as pl
from jax.experimental.pallas import tpu as pltpu
from jax.experimental.pallas import tpu_sc as plsc

N = 10000
E = 160000
D = 256
NR = 16

NC = 2
NS = 16
NW = NC * NS

CHUNK = 1024
EP = 163840
EPW = EP // NS
NCHUNK = EPW // CHUNK
CBUF = 163840
PAD_NODE = 10240

OWN = 320
NOUT = NW * OWN
ACCR = 328
DUMP = 324
FL = 64
CB = 208
SCH = 2048
NSCH = EP // SCH
NSG = SCH // 128

BLK = 400

_SC_PARAMS = pltpu.CompilerParams(needs_layout_passes=False)


_DIMS = (((1,), (0,)), ((), ()))


def _dot(a, b):
    a_hi = a.astype(jnp.bfloat16)
    a_lo = (a - a_hi.astype(jnp.float32)).astype(jnp.bfloat16)
    b_hi = b.astype(jnp.bfloat16)
    b_lo = (b - b_hi.astype(jnp.float32)).astype(jnp.bfloat16)

    def d(u, v):
        return lax.dot_general(u, v, _DIMS,
                               preferred_element_type=jnp.float32)

    return d(a_hi, b_hi) + d(a_hi, b_lo) + d(a_lo, b_hi)


def _sc_counts(nodes2, rel_p):
    mesh = plsc.VectorSubcoreMesh(core_axis_name="c", subcore_axis_name="s")

    @functools.partial(
        pl.kernel,
        out_type=jax.ShapeDtypeStruct((2, CBUF), jnp.float32),
        mesh=mesh,
        compiler_params=_SC_PARAMS,
        scratch_types=[
            pltpu.VMEM((CHUNK,), jnp.int32),
            pltpu.VMEM((CHUNK,), jnp.int32),
            pltpu.VMEM((CHUNK,), jnp.int32),
            pltpu.VMEM((CHUNK,), jnp.int32),
            pltpu.VMEM((8, 128), jnp.int32),
            pltpu.VMEM((8, 128), jnp.int32),
            pltpu.VMEM((128,), jnp.float32),
            pltpu.VMEM((2048,), jnp.float32),
            pltpu.VMEM_SHARED((CBUF,), jnp.float32),
            pltpu.SemaphoreType.DMA,
            pltpu.SemaphoreType.DMA,
            pltpu.SemaphoreType.DMA,
            pltpu.SemaphoreType.DMA,
        ],
    )
    def k(nodes_hbm, rel_hbm, out_hbm, na_v, ra_v, nb_v, rb_v, ia_v, ib_v,
          ones_v, zeros_v, acc_sh, semLA, semLB, semSA, semSB):
        cid = lax.axis_index("c")
        sid = lax.axis_index("s")
        col16c = lax.iota(jnp.int32, 16)

        @pl.loop(0, 128, step=16)
        def _(i):
            ones_v[pl.ds(i, 16)] = jnp.full((16,), 1.0, jnp.float32)

        @pl.loop(0, 2048, step=16)
        def _(i):
            zeros_v[pl.ds(i, 16)] = jnp.zeros((16,), jnp.float32)

        @pl.loop(0, CBUF // NS, step=2048)
        def _(j):
            pltpu.sync_copy(zeros_v,
                            acc_sh.at[pl.ds(sid * (CBUF // NS) + j, 2048)])

        plsc.subcore_barrier()

        def load(ci, n_v, r_v, semL):
            eb = sid * EPW + ci * CHUNK
            pltpu.async_copy(nodes_hbm.at[cid, pl.ds(eb, CHUNK)], n_v, semL)
            pltpu.async_copy(rel_hbm.at[pl.ds(eb, CHUNK)], r_v, semL)

        def wait_load(ci, n_v, r_v, semL):
            eb = sid * EPW + ci * CHUNK
            pltpu.make_async_copy(nodes_hbm.at[cid, pl.ds(eb, CHUNK)], n_v,
                                  semL).wait()
            pltpu.make_async_copy(rel_hbm.at[pl.ds(eb, CHUNK)], r_v,
                                  semL).wait()

        def scatter(n_v, r_v, i2_v, semS):
            @pl.loop(0, 8)
            def _(r):
                @pl.loop(0, 128, step=16)
                def _(c):
                    nv = n_v[pl.ds(r * 128 + c, 16)]
                    rv = r_v[pl.ds(r * 128 + c, 16)]
                    i2_v[r, pl.ds(c, 16)] = nv * NR + rv
            for r in range(8):
                pltpu.async_copy(ones_v, acc_sh.at[i2_v.at[r]], semS,
                                 add=True)

        def drain_scatter(i2_v, semS):
            for r in range(8):
                pltpu.make_async_copy(ones_v, acc_sh.at[i2_v.at[r]],
                                      semS).wait()

        load(0, na_v, ra_v, semLA)

        for i2_v, semS in ((ia_v, semSA), (ib_v, semSB)):
            @pl.loop(0, 8)
            def _(r):
                @pl.loop(0, 128, step=16)
                def _(c):
                    i2_v[r, pl.ds(c, 16)] = (
                        N * NR + ((sid * 1024 + r * 128 + c + col16c) & 2047))
            for r in range(8):
                pltpu.async_copy(zeros_v.at[pl.ds(0, 128)],
                                 acc_sh.at[i2_v.at[r]], semS, add=True)

        def step(p, carry):
            load(2 * p + 1, nb_v, rb_v, semLB)
            wait_load(2 * p, na_v, ra_v, semLA)
            drain_scatter(ia_v, semSA)
            scatter(na_v, ra_v, ia_v, semSA)
            nxt = jnp.minimum(2 * p + 2, NCHUNK - 1)
            load(nxt, na_v, ra_v, semLA)
            wait_load(2 * p + 1, nb_v, rb_v, semLB)
            drain_scatter(ib_v, semSB)
            scatter(nb_v, rb_v, ib_v, semSB)
            return carry

        lax.fori_loop(0, NCHUNK // 2, step, jnp.int32(0))
        wait_load(NCHUNK - 1, na_v, ra_v, semLA)
        drain_scatter(ia_v, semSA)
        drain_scatter(ib_v, semSB)

        plsc.subcore_barrier()

        @pl.loop(0, CBUF // NS, step=2048)
        def _(j):
            off = sid * (CBUF // NS) + j
            pltpu.sync_copy(acc_sh.at[pl.ds(off, 2048)],
                            out_hbm.at[cid, pl.ds(off, 2048)])

    return k(nodes2, rel_p)


def _sc_agg(h_aug, epk):
    mesh = plsc.VectorSubcoreMesh(core_axis_name="c", subcore_axis_name="s")

    @functools.partial(
        pl.kernel,
        out_type=jax.ShapeDtypeStruct((NOUT, D), jnp.float32),
        mesh=mesh,
        compiler_params=_SC_PARAMS,
        scratch_types=[
            pltpu.VMEM((2, SCH), jnp.int32),
            pltpu.VMEM((2, SCH), jnp.int32),
            pltpu.VMEM((CB,), jnp.int32),
            pltpu.VMEM((CB,), jnp.int32),
            pltpu.VMEM((FL,), jnp.int32),
            pltpu.VMEM((FL,), jnp.int32),
            pltpu.VMEM((FL,), jnp.int32),
            pltpu.VMEM((FL,), jnp.int32),
            pltpu.VMEM((FL, D), jnp.float32),
            pltpu.VMEM((FL, D), jnp.float32),
            pltpu.VMEM((ACCR, D), jnp.float32),
            pltpu.SemaphoreType.DMA,
            pltpu.SemaphoreType.DMA,
            pltpu.SemaphoreType.DMA,
            pltpu.SemaphoreType.DMA,
        ],
    )
    def k(h_hbm, epk_hbm, out_hbm, e2a, e2b, psrc_v, pdl_v,
          fsrc0, fdl0, fsrc1, fdl1, gr0, gr1, acc_v, sem0, sem1,
          semA, semB):
        cid = lax.axis_index("c")
        sid = lax.axis_index("s")
        w = sid * NC + cid
        rbase = w * OWN
        col16 = lax.iota(jnp.int32, 16)
        bufs = ((fsrc0, fdl0, gr0, sem0), (fsrc1, fdl1, gr1, sem1))

        @pl.loop(0, ACCR)
        def _(r):
            for j in range(0, D, 16):
                acc_v[r, pl.ds(j, 16)] = jnp.zeros((16,), jnp.float32)

        def accum(fdl_b, gr_b):
            def acc_body(i, carry):
                e0 = i * 4
                for e in (e0, e0 + 1, e0 + 2, e0 + 3):
                    eb = jnp.full((16,), e, jnp.int32)
                    row16 = plsc.load_gather(fdl_b, [eb])
                    for kk in range(0, D, 16):
                        vals = gr_b[e, pl.ds(kk, 16)]
                        plsc.addupdate_scatter(acc_v, [row16, col16 + kk],
                                               vals)
                return carry

            lax.fori_loop(0, FL // 4, acc_body, 0)

        def flush_with(par):
            fsrc_a, fdl_a, gr_a, sem_a = bufs[par]
            fsrc_b, fdl_b, gr_b, sem_b = bufs[1 - par]

            def go(off):
                for j in range(0, FL, 16):
                    pk = psrc_v[pl.ds(j, 16)]
                    fsrc_a[pl.ds(j, 16)] = jax.lax.shift_right_logical(pk, 9)
                    fdl_a[pl.ds(j, 16)] = pk & 511
                pltpu.async_copy(h_hbm.at[fsrc_a], gr_a, sem_a)
                pltpu.make_async_copy(h_hbm.at[fsrc_b], gr_b, sem_b).wait()
                accum(fdl_b, gr_b)
                for jj in range(0, 2 * FL, 16):
                    psrc_v[pl.ds(jj, 16)] = psrc_v[pl.ds(FL + jj, 16)]
                return off - FL

            return go

        def flush_sel(op):
            off, par = op
            off = lax.cond(par == 0, flush_with(0), flush_with(1), off)
            return (off, 1 - par)

        def process(buf):
            def sg_body(sg, op):
                off, par = op
                b0 = sg * 128
                ss, dls, ms, css, cnts = [], [], [], [], []
                for j in range(8):
                    s16 = buf[0, pl.ds(b0 + j * 16, 16)]
                    d16 = buf[1, pl.ds(b0 + j * 16, 16)]
                    dl = d16 - rbase
                    m = (dl >= 0) & (dl < OWN)
                    m32 = m.astype(jnp.int32)
                    ss.append(s16)
                    dls.append(dl)
                    ms.append(m)
                    css.append(plsc.cumsum(m32))
                    cnts.append(jnp.sum(m32))
                o = off
                for j in range(8):
                    pos = o + css[j] - 1
                    plsc.store_scatter(psrc_v, [pos],
                                       ss[j] * 512 + dls[j], mask=ms[j])
                    o = o + cnts[j]
                op2 = lax.cond(o >= FL, flush_sel, lambda t: t, (o, par))
                return lax.cond(op2[0] >= FL, flush_sel, lambda t: t, op2)

            return lambda op: lax.fori_loop(0, NSG, sg_body, op)

        def pair(p, op):
            pltpu.async_copy(epk_hbm.at[2 * p + 1], e2b, semB)
            pltpu.make_async_copy(epk_hbm.at[2 * p], e2a, semA).wait()
            op = process(e2a)(op)
            nxt = jnp.minimum(2 * p + 2, NSCH - 1)
            pltpu.async_copy(epk_hbm.at[nxt], e2a, semA)
            pltpu.make_async_copy(epk_hbm.at[2 * p + 1], e2b, semB).wait()
            return process(e2b)(op)

        def fill_dump(off_c):
            def pad_body(t, o):
                psrc_v[pl.ds(o + t * 16, 16)] = (
                    (rbase + t * 16 + col16) * 512 + DUMP)
                return o

            lax.fori_loop(0, 5, pad_body, off_c)

        for j in range(0, FL, 16):
            fsrc1[pl.ds(j, 16)] = rbase + j + col16
            fdl1[pl.ds(j, 16)] = jnp.full((16,), DUMP, jnp.int32)
        pltpu.async_copy(h_hbm.at[fsrc1], gr1, sem1)

        pltpu.async_copy(epk_hbm.at[0], e2a, semA)
        op = lax.fori_loop(0, NSCH // 2, pair,
                           (jnp.int32(0), jnp.int32(0)))
        pltpu.make_async_copy(epk_hbm.at[NSCH - 1], e2a, semA).wait()

        off, par = op
        fill_dump(off)
        off, par = flush_sel((off, par))
        fill_dump(jnp.int32(0))
        off, par = flush_sel((off, par))

        def drain_with(par_c):
            def go(z):
                fsrc_b, _, gr_b, sem_b = bufs[1 - par_c]
                pltpu.make_async_copy(h_hbm.at[fsrc_b], gr_b, sem_b).wait()
                return z

            return go

        lax.cond(par == 0, drain_with(0), drain_with(1), jnp.int32(0))

        plsc.subcore_barrier()
        pltpu.sync_copy(acc_v.at[pl.ds(0, OWN)],
                        out_hbm.at[pl.ds(rbase, OWN)])

    return k(h_aug, epk)


def _tc_dense0(x, W0, b0):
    def body(x_ref, w_ref, b_ref, h_ref):
        h_ref[...] = jnp.tanh(_dot(x_ref[...], w_ref[...]) + b_ref[...])

    return pl.pallas_call(
        body,
        grid=(N // BLK,),
        in_specs=[
            pl.BlockSpec((BLK, D), lambda i: (i, 0)),
            pl.BlockSpec((D, D), lambda i: (0, 0)),
            pl.BlockSpec((1, D), lambda i: (0, 0)),
        ],
        out_specs=pl.BlockSpec((BLK, D), lambda i: (i, 0)),
        out_shape=jax.ShapeDtypeStruct((N, D), jnp.float32),
    )(x, W0, b0.reshape(1, D))


def _tc_haug(h, c_src, emb_rel):
    def body(h_ref, c_ref, e_ref, ha_ref):
        ha_ref[...] = h_ref[...] + _dot(c_ref[...], e_ref[...])

    return pl.pallas_call(
        body,
        grid=(N // BLK,),
        in_specs=[
            pl.BlockSpec((BLK, D), lambda i: (i, 0)),
            pl.BlockSpec((BLK, NR), lambda i: (i, 0)),
            pl.BlockSpec((NR, D), lambda i: (0, 0)),
        ],
        out_specs=pl.BlockSpec((BLK, D), lambda i: (i, 0)),
        out_shape=jax.ShapeDtypeStruct((N, D), jnp.float32),
    )(h, c_src, emb_rel)


def _tc_output(agg, c_dst, emb_rel, h, W1, b1, W2, b2):
    def body(a_ref, c_ref, e_ref, h_ref, w1_ref, b1_ref, w2_ref, b2_ref,
             o_ref):
        cb = c_ref[...]
        deg = jnp.maximum(jnp.sum(cb, axis=1, keepdims=True), 1.0)
        t = a_ref[...] / deg
        u = jnp.tanh(_dot(t, w1_ref[...]) + b1_ref[...] + h_ref[...]
                     + _dot(cb, e_ref[...]))
        o_ref[...] = _dot(u, w2_ref[...]) + b2_ref[...]

    return pl.pallas_call(
        body,
        grid=(N // BLK,),
        in_specs=[
            pl.BlockSpec((BLK, D), lambda i: (i, 0)),
            pl.BlockSpec((BLK, NR), lambda i: (i, 0)),
            pl.BlockSpec((NR, D), lambda i: (0, 0)),
            pl.BlockSpec((BLK, D), lambda i: (i, 0)),
            pl.BlockSpec((D, D), lambda i: (0, 0)),
            pl.BlockSpec((1, D), lambda i: (0, 0)),
            pl.BlockSpec((D, D), lambda i: (0, 0)),
            pl.BlockSpec((1, D), lambda i: (0, 0)),
        ],
        out_specs=pl.BlockSpec((BLK, D), lambda i: (i, 0)),
        out_shape=jax.ShapeDtypeStruct((N, D), jnp.float32),
    )(agg, c_dst, emb_rel, h, W1, b1.reshape(1, D), W2, b2.reshape(1, D))


def kernel(x, edge_index, rel_ids, emb_rel, W0, b0, W1, b1, W2, b2):
    src = edge_index[0].astype(jnp.int32)
    dst = edge_index[1].astype(jnp.int32)
    rel = rel_ids.astype(jnp.int32)

    pad = EP - E
    src_cnt = jnp.concatenate([src, jnp.full((pad,), N, jnp.int32)])
    dst_cnt = jnp.concatenate([dst, jnp.full((pad,), N, jnp.int32)])
    dst_p = jnp.concatenate([dst, jnp.full((pad,), PAD_NODE, jnp.int32)])
    src_g = jnp.concatenate([src, jnp.zeros((pad,), jnp.int32)])
    rel_p = jnp.concatenate([rel, jnp.zeros((pad,), jnp.int32)])
    nodes2 = jnp.stack([src_cnt, dst_cnt])

    epk = jnp.stack([src_g.reshape(NSCH, SCH), dst_p.reshape(NSCH, SCH)],
                    axis=1)

    counts = _sc_counts(nodes2, rel_p)
    c_src = counts[0, : N * NR].reshape(N, NR)
    c_dst = counts[1, : N * NR].reshape(N, NR)

    h = _tc_dense0(x, W0, b0)
    h_aug = _tc_haug(h, c_src, emb_rel)
    agg = _sc_agg(h_aug, epk)[:N]
    return _tc_output(agg, c_dst, emb_rel, h, W1, b1, W2, b2)


# speedup vs baseline: 1.0061x; 1.0061x over previous
import functools

import jax
import jax.numpy as jnp
from jax import lax
from jax.experimental import pallas as pl
from jax.experimental.pallas import tpu as pltpu
from jax.experimental.pallas import tpu_sc as plsc

N = 10000
E = 160000
D = 256
NR = 16

NC = 2
NS = 16
NW = NC * NS

CHUNK = 1024
EP = 163840
EPW = EP // NS
NCHUNK = EPW // CHUNK
CBUF = 163840
PAD_NODE = 10240

OWN = 320
NOUT = NW * OWN
ACCR = 328
DUMP = 324
FL = 64
CB = 208
SCH = 2048
NSCH = EP // SCH
NSG = SCH // 128

BLK = 400

_SC_PARAMS = pltpu.CompilerParams(needs_layout_passes=False)


_DIMS = (((1,), (0,)), ((), ()))


def _dot(a, b):
    a_hi = a.astype(jnp.bfloat16)
    a_lo = (a - a_hi.astype(jnp.float32)).astype(jnp.bfloat16)
    b_hi = b.astype(jnp.bfloat16)
    b_lo = (b - b_hi.astype(jnp.float32)).astype(jnp.bfloat16)

    def d(u, v):
        return lax.dot_general(u, v, _DIMS,
                               preferred_element_type=jnp.float32)

    return d(a_hi, b_hi) + d(a_hi, b_lo) + d(a_lo, b_hi)


def _sc_counts(nodes2, rel_p):
    mesh = plsc.VectorSubcoreMesh(core_axis_name="c", subcore_axis_name="s")

    @functools.partial(
        pl.kernel,
        out_type=jax.ShapeDtypeStruct((2, CBUF), jnp.float32),
        mesh=mesh,
        compiler_params=_SC_PARAMS,
        scratch_types=[
            pltpu.VMEM((CHUNK,), jnp.int32),
            pltpu.VMEM((CHUNK,), jnp.int32),
            pltpu.VMEM((CHUNK,), jnp.int32),
            pltpu.VMEM((CHUNK,), jnp.int32),
            pltpu.VMEM((8, 128), jnp.int32),
            pltpu.VMEM((8, 128), jnp.int32),
            pltpu.VMEM((128,), jnp.float32),
            pltpu.VMEM((2048,), jnp.float32),
            pltpu.VMEM_SHARED((CBUF,), jnp.float32),
            pltpu.SemaphoreType.DMA,
            pltpu.SemaphoreType.DMA,
            pltpu.SemaphoreType.DMA,
            pltpu.SemaphoreType.DMA,
        ],
    )
    def k(nodes_hbm, rel_hbm, out_hbm, na_v, ra_v, nb_v, rb_v, ia_v, ib_v,
          ones_v, zeros_v, acc_sh, semLA, semLB, semSA, semSB):
        cid = lax.axis_index("c")
        sid = lax.axis_index("s")
        col16c = lax.iota(jnp.int32, 16)

        @pl.loop(0, 128, step=16)
        def _(i):
            ones_v[pl.ds(i, 16)] = jnp.full((16,), 1.0, jnp.float32)

        @pl.loop(0, 2048, step=16)
        def _(i):
            zeros_v[pl.ds(i, 16)] = jnp.zeros((16,), jnp.float32)

        @pl.loop(0, CBUF // NS, step=2048)
        def _(j):
            pltpu.sync_copy(zeros_v,
                            acc_sh.at[pl.ds(sid * (CBUF // NS) + j, 2048)])

        plsc.subcore_barrier()

        def load(ci, n_v, r_v, semL):
            eb = sid * EPW + ci * CHUNK
            pltpu.async_copy(nodes_hbm.at[cid, pl.ds(eb, CHUNK)], n_v, semL)
            pltpu.async_copy(rel_hbm.at[pl.ds(eb, CHUNK)], r_v, semL)

        def wait_load(ci, n_v, r_v, semL):
            eb = sid * EPW + ci * CHUNK
            pltpu.make_async_copy(nodes_hbm.at[cid, pl.ds(eb, CHUNK)], n_v,
                                  semL).wait()
            pltpu.make_async_copy(rel_hbm.at[pl.ds(eb, CHUNK)], r_v,
                                  semL).wait()

        def scatter(n_v, r_v, i2_v, semS):
            @pl.loop(0, 8)
            def _(r):
                @pl.loop(0, 128, step=16)
                def _(c):
                    nv = n_v[pl.ds(r * 128 + c, 16)]
                    rv = r_v[pl.ds(r * 128 + c, 16)]
                    i2_v[r, pl.ds(c, 16)] = nv * NR + rv
            for r in range(8):
                pltpu.async_copy(ones_v, acc_sh.at[i2_v.at[r]], semS,
                                 add=True)

        def drain_scatter(i2_v, semS):
            for r in range(8):
                pltpu.make_async_copy(ones_v, acc_sh.at[i2_v.at[r]],
                                      semS).wait()

        load(0, na_v, ra_v, semLA)

        for i2_v, semS in ((ia_v, semSA), (ib_v, semSB)):
            @pl.loop(0, 8)
            def _(r):
                @pl.loop(0, 128, step=16)
                def _(c):
                    i2_v[r, pl.ds(c, 16)] = (
                        N * NR + ((sid * 1024 + r * 128 + c + col16c) & 2047))
            for r in range(8):
                pltpu.async_copy(zeros_v.at[pl.ds(0, 128)],
                                 acc_sh.at[i2_v.at[r]], semS, add=True)

        def step(p, carry):
            load(2 * p + 1, nb_v, rb_v, semLB)
            wait_load(2 * p, na_v, ra_v, semLA)
            drain_scatter(ia_v, semSA)
            scatter(na_v, ra_v, ia_v, semSA)
            nxt = jnp.minimum(2 * p + 2, NCHUNK - 1)
            load(nxt, na_v, ra_v, semLA)
            wait_load(2 * p + 1, nb_v, rb_v, semLB)
            drain_scatter(ib_v, semSB)
            scatter(nb_v, rb_v, ib_v, semSB)
            return carry

        lax.fori_loop(0, NCHUNK // 2, step, jnp.int32(0))
        wait_load(NCHUNK - 1, na_v, ra_v, semLA)
        drain_scatter(ia_v, semSA)
        drain_scatter(ib_v, semSB)

        plsc.subcore_barrier()

        @pl.loop(0, CBUF // NS, step=2048)
        def _(j):
            off = sid * (CBUF // NS) + j
            pltpu.sync_copy(acc_sh.at[pl.ds(off, 2048)],
                            out_hbm.at[cid, pl.ds(off, 2048)])

    return k(nodes2, rel_p)


def _sc_agg(h_aug, epk):
    mesh = plsc.VectorSubcoreMesh(core_axis_name="c", subcore_axis_name="s")

    @functools.partial(
        pl.kernel,
        out_type=jax.ShapeDtypeStruct((NOUT, D), jnp.float32),
        mesh=mesh,
        compiler_params=_SC_PARAMS,
        scratch_types=[
            pltpu.VMEM((2, SCH), jnp.int32),
            pltpu.VMEM((2, SCH), jnp.int32),
            pltpu.VMEM((CB,), jnp.int32),
            pltpu.VMEM((CB,), jnp.int32),
            pltpu.VMEM((FL,), jnp.int32),
            pltpu.VMEM((FL,), jnp.int32),
            pltpu.VMEM((FL,), jnp.int32),
            pltpu.VMEM((FL,), jnp.int32),
            pltpu.VMEM((FL, D), jnp.float32),
            pltpu.VMEM((FL, D), jnp.float32),
            pltpu.VMEM((ACCR, D), jnp.float32),
            pltpu.SemaphoreType.DMA,
            pltpu.SemaphoreType.DMA,
            pltpu.SemaphoreType.DMA,
            pltpu.SemaphoreType.DMA,
        ],
    )
    def k(h_hbm, epk_hbm, out_hbm, e2a, e2b, psrc_v, pdl_v,
          fsrc0, fdl0, fsrc1, fdl1, gr0, gr1, acc_v, sem0, sem1,
          semA, semB):
        cid = lax.axis_index("c")
        sid = lax.axis_index("s")
        w = sid * NC + cid
        rbase = w * OWN
        col16 = lax.iota(jnp.int32, 16)
        bufs = ((fsrc0, fdl0, gr0, sem0), (fsrc1, fdl1, gr1, sem1))

        @pl.loop(0, ACCR)
        def _(r):
            for j in range(0, D, 16):
                acc_v[r, pl.ds(j, 16)] = jnp.zeros((16,), jnp.float32)

        def accum(fdl_b, gr_b):
            def acc_body(i, carry):
                e0 = i * 2
                for e in (e0, e0 + 1):
                    eb = jnp.full((16,), e, jnp.int32)
                    row16 = plsc.load_gather(fdl_b, [eb])
                    for kk in range(0, D, 16):
                        vals = gr_b[e, pl.ds(kk, 16)]
                        plsc.addupdate_scatter(acc_v, [row16, col16 + kk],
                                               vals)
                return carry

            lax.fori_loop(0, FL // 2, acc_body, 0)

        def flush_with(par):
            fsrc_a, fdl_a, gr_a, sem_a = bufs[par]
            fsrc_b, fdl_b, gr_b, sem_b = bufs[1 - par]

            def go(off):
                for j in range(0, FL, 16):
                    pk = psrc_v[pl.ds(j, 16)]
                    fsrc_a[pl.ds(j, 16)] = jax.lax.shift_right_logical(pk, 9)
                    fdl_a[pl.ds(j, 16)] = pk & 511
                pltpu.async_copy(h_hbm.at[fsrc_a], gr_a, sem_a)
                pltpu.make_async_copy(h_hbm.at[fsrc_b], gr_b, sem_b).wait()
                accum(fdl_b, gr_b)
                for jj in range(0, 2 * FL, 16):
                    psrc_v[pl.ds(jj, 16)] = psrc_v[pl.ds(FL + jj, 16)]
                return off - FL

            return go

        def flush_sel(op):
            off, par = op
            off = lax.cond(par == 0, flush_with(0), flush_with(1), off)
            return (off, 1 - par)

        def process(buf):
            def sg_body(sg, op):
                off, par = op
                b0 = sg * 128
                ss, dls, ms, css, cnts = [], [], [], [], []
                for j in range(8):
                    s16 = buf[0, pl.ds(b0 + j * 16, 16)]
                    d16 = buf[1, pl.ds(b0 + j * 16, 16)]
                    dl = d16 - rbase
                    m = (dl >= 0) & (dl < OWN)
                    m32 = m.astype(jnp.int32)
                    ss.append(s16)
                    dls.append(dl)
                    ms.append(m)
                    css.append(plsc.cumsum(m32))
                    cnts.append(jnp.sum(m32))
                o = off
                for j in range(8):
                    pos = o + css[j] - 1
                    plsc.store_scatter(psrc_v, [pos],
                                       ss[j] * 512 + dls[j], mask=ms[j])
                    o = o + cnts[j]
                op2 = lax.cond(o >= FL, flush_sel, lambda t: t, (o, par))
                return lax.cond(op2[0] >= FL, flush_sel, lambda t: t, op2)

            return lambda op: lax.fori_loop(0, NSG, sg_body, op)

        def pair(p, op):
            pltpu.async_copy(epk_hbm.at[2 * p + 1], e2b, semB)
            pltpu.make_async_copy(epk_hbm.at[2 * p], e2a, semA).wait()
            op = process(e2a)(op)
            nxt = jnp.minimum(2 * p + 2, NSCH - 1)
            pltpu.async_copy(epk_hbm.at[nxt], e2a, semA)
            pltpu.make_async_copy(epk_hbm.at[2 * p + 1], e2b, semB).wait()
            return process(e2b)(op)

        def fill_dump(off_c):
            def pad_body(t, o):
                psrc_v[pl.ds(o + t * 16, 16)] = (
                    (rbase + t * 16 + col16) * 512 + DUMP)
                return o

            lax.fori_loop(0, 5, pad_body, off_c)

        for j in range(0, FL, 16):
            fsrc1[pl.ds(j, 16)] = rbase + j + col16
            fdl1[pl.ds(j, 16)] = jnp.full((16,), DUMP, jnp.int32)
        pltpu.async_copy(h_hbm.at[fsrc1], gr1, sem1)

        pltpu.async_copy(epk_hbm.at[0], e2a, semA)
        op = lax.fori_loop(0, NSCH // 2, pair,
                           (jnp.int32(0), jnp.int32(0)))
        pltpu.make_async_copy(epk_hbm.at[NSCH - 1], e2a, semA).wait()

        off, par = op
        fill_dump(off)
        off, par = flush_sel((off, par))
        fill_dump(jnp.int32(0))
        off, par = flush_sel((off, par))

        def drain_with(par_c):
            def go(z):
                fsrc_b, _, gr_b, sem_b = bufs[1 - par_c]
                pltpu.make_async_copy(h_hbm.at[fsrc_b], gr_b, sem_b).wait()
                return z

            return go

        lax.cond(par == 0, drain_with(0), drain_with(1), jnp.int32(0))

        plsc.subcore_barrier()
        pltpu.sync_copy(acc_v.at[pl.ds(0, OWN)],
                        out_hbm.at[pl.ds(rbase, OWN)])

    return k(h_aug, epk)


def _tc_dense0(x, W0, b0):
    def body(x_ref, w_ref, b_ref, h_ref):
        h_ref[...] = jnp.tanh(_dot(x_ref[...], w_ref[...]) + b_ref[...])

    return pl.pallas_call(
        body,
        grid=(N // BLK,),
        in_specs=[
            pl.BlockSpec((BLK, D), lambda i: (i, 0)),
            pl.BlockSpec((D, D), lambda i: (0, 0)),
            pl.BlockSpec((1, D), lambda i: (0, 0)),
        ],
        out_specs=pl.BlockSpec((BLK, D), lambda i: (i, 0)),
        out_shape=jax.ShapeDtypeStruct((N, D), jnp.float32),
    )(x, W0, b0.reshape(1, D))


def _tc_haug(h, c_src, emb_rel):
    def body(h_ref, c_ref, e_ref, ha_ref):
        ha_ref[...] = h_ref[...] + _dot(c_ref[...], e_ref[...])

    return pl.pallas_call(
        body,
        grid=(N // BLK,),
        in_specs=[
            pl.BlockSpec((BLK, D), lambda i: (i, 0)),
            pl.BlockSpec((BLK, NR), lambda i: (i, 0)),
            pl.BlockSpec((NR, D), lambda i: (0, 0)),
        ],
        out_specs=pl.BlockSpec((BLK, D), lambda i: (i, 0)),
        out_shape=jax.ShapeDtypeStruct((N, D), jnp.float32),
    )(h, c_src, emb_rel)


def _tc_output(agg, c_dst, emb_rel, h, W1, b1, W2, b2):
    def body(a_ref, c_ref, e_ref, h_ref, w1_ref, b1_ref, w2_ref, b2_ref,
             o_ref):
        cb = c_ref[...]
        deg = jnp.maximum(jnp.sum(cb, axis=1, keepdims=True), 1.0)
        t = a_ref[...] / deg
        u = jnp.tanh(_dot(t, w1_ref[...]) + b1_ref[...] + h_ref[...]
                     + _dot(cb, e_ref[...]))
        o_ref[...] = _dot(u, w2_ref[...]) + b2_ref[...]

    return pl.pallas_call(
        body,
        grid=(N // BLK,),
        in_specs=[
            pl.BlockSpec((BLK, D), lambda i: (i, 0)),
            pl.BlockSpec((BLK, NR), lambda i: (i, 0)),
            pl.BlockSpec((NR, D), lambda i: (0, 0)),
            pl.BlockSpec((BLK, D), lambda i: (i, 0)),
            pl.BlockSpec((D, D), lambda i: (0, 0)),
            pl.BlockSpec((1, D), lambda i: (0, 0)),
            pl.BlockSpec((D, D), lambda i: (0, 0)),
            pl.BlockSpec((1, D), lambda i: (0, 0)),
        ],
        out_specs=pl.BlockSpec((BLK, D), lambda i: (i, 0)),
        out_shape=jax.ShapeDtypeStruct((N, D), jnp.float32),
    )(agg, c_dst, emb_rel, h, W1, b1.reshape(1, D), W2, b2.reshape(1, D))


def kernel(x, edge_index, rel_ids, emb_rel, W0, b0, W1, b1, W2, b2):
    src = edge_index[0].astype(jnp.int32)
    dst = edge_index[1].astype(jnp.int32)
    rel = rel_ids.astype(jnp.int32)

    pad = EP - E
    src_cnt = jnp.concatenate([src, jnp.full((pad,), N, jnp.int32)])
    dst_cnt = jnp.concatenate([dst, jnp.full((pad,), N, jnp.int32)])
    dst_p = jnp.concatenate([dst, jnp.full((pad,), PAD_NODE, jnp.int32)])
    src_g = jnp.concatenate([src, jnp.zeros((pad,), jnp.int32)])
    rel_p = jnp.concatenate([rel, jnp.zeros((pad,), jnp.int32)])
    nodes2 = jnp.stack([src_cnt, dst_cnt])

    epk = jnp.stack([src_g.reshape(NSCH, SCH), dst_p.reshape(NSCH, SCH)],
                    axis=1)

    counts = _sc_counts(nodes2, rel_p)
    c_src = counts[0, : N * NR].reshape(N, NR)
    c_dst = counts[1, : N * NR].reshape(N, NR)

    h = _tc_dense0(x, W0, b0)
    h_aug = _tc_haug(h, c_src, emb_rel)
    agg = _sc_agg(h_aug, epk)[:N]
    return _tc_output(agg, c_dst, emb_rel, h, W1, b1, W2, b2)
